# baseline (speedup 1.0000x reference)
import jax
import jax.numpy as jnp
from jax import lax
from jax.experimental import pallas as pl
from jax.experimental.pallas import tpu as pltpu

T = 2048
D = 1024
E = 8
TOP_K = 2
NN = 4
DN = D // NN


def _moe_dense_body(x_ref, wg_ref, bg_ref, w_ref, b_ref,
                    out_ref, idx_ref, comb_ref, xb_ref):
    n = pl.program_id(0)
    e = pl.program_id(1)

    @pl.when((n == 0) & (e == 0))
    def _gate():
        logits = jnp.dot(x_ref[...], wg_ref[...],
                         preferred_element_type=jnp.float32) + bg_ref[...]
        col = lax.broadcasted_iota(jnp.int32, (T, E), 1)
        m1 = jnp.max(logits, axis=1, keepdims=True)
        i1 = jnp.min(jnp.where(logits == m1, col, E), axis=1, keepdims=True)
        masked = jnp.where(col == i1, -jnp.inf, logits)
        m2 = jnp.max(masked, axis=1, keepdims=True)
        i2 = jnp.min(jnp.where(masked == m2, col, E), axis=1, keepdims=True)
        r = jnp.exp(m2 - m1)
        w1 = 1.0 / (1.0 + r)
        w2 = r / (1.0 + r)
        comb_ref[...] = (jnp.where(col == i1, w1, 0.0)
                         + jnp.where(col == i2, w2, 0.0))
        idx_ref[...] = jnp.concatenate([i1, i2], axis=1)
        xb_ref[...] = x_ref[...].astype(jnp.bfloat16)

    @pl.when(e == 0)
    def _init():
        out_ref[...] = jnp.dot(comb_ref[...], b_ref[...],
                               preferred_element_type=jnp.float32)

    ce = jnp.sum(
        comb_ref[...] * (lax.broadcasted_iota(jnp.int32, (T, E), 1) == e),
        axis=1, keepdims=True)
    acc = jnp.dot(xb_ref[...], w_ref[0].astype(jnp.bfloat16),
                  preferred_element_type=jnp.float32)
    out_ref[...] += ce * acc


@jax.jit
def kernel(x, Wg, bg, W, b):
    bg2 = bg.reshape(1, E)
    grid = (NN, E)
    out, idx = pl.pallas_call(
        _moe_dense_body,
        grid=grid,
        in_specs=[
            pl.BlockSpec((T, D), lambda n, e: (0, 0)),
            pl.BlockSpec((D, E), lambda n, e: (0, 0)),
            pl.BlockSpec((1, E), lambda n, e: (0, 0)),
            pl.BlockSpec((1, D, DN), lambda n, e: (e, 0, n)),
            pl.BlockSpec((E, DN), lambda n, e: (0, n)),
        ],
        out_specs=[
            pl.BlockSpec((T, DN), lambda n, e: (0, n)),
            pl.BlockSpec((T, TOP_K), lambda n, e: (0, 0)),
        ],
        out_shape=[
            jax.ShapeDtypeStruct((T, D), jnp.float32),
            jax.ShapeDtypeStruct((T, TOP_K), jnp.int32),
        ],
        scratch_shapes=[
            pltpu.VMEM((T, E), jnp.float32),
            pltpu.VMEM((T, D), jnp.bfloat16),
        ],
        compiler_params=pltpu.CompilerParams(
            dimension_semantics=("arbitrary", "arbitrary"),
        ),
    )(x, Wg, bg2, W, b)
    return out, idx


# speedup vs baseline: 1.0856x; 1.0856x over previous
import jax
import jax.numpy as jnp
from jax import lax
from jax.experimental import pallas as pl
from jax.experimental.pallas import tpu as pltpu

T = 2048
D = 1024
E = 8
TOP_K = 2
NN = 2
DN = D // NN


def _moe_dense_body(x_ref, wg_ref, bg_ref, w_ref, b_ref,
                    out_ref, idx_ref, comb_ref, xb_ref):
    n = pl.program_id(0)
    e = pl.program_id(1)

    @pl.when((n == 0) & (e == 0))
    def _gate():
        logits = jnp.dot(x_ref[...], wg_ref[...],
                         preferred_element_type=jnp.float32) + bg_ref[...]
        col = lax.broadcasted_iota(jnp.int32, (T, E), 1)
        m1 = jnp.max(logits, axis=1, keepdims=True)
        i1 = jnp.min(jnp.where(logits == m1, col, E), axis=1, keepdims=True)
        masked = jnp.where(col == i1, -jnp.inf, logits)
        m2 = jnp.max(masked, axis=1, keepdims=True)
        i2 = jnp.min(jnp.where(masked == m2, col, E), axis=1, keepdims=True)
        r = jnp.exp(m2 - m1)
        w1 = 1.0 / (1.0 + r)
        w2 = r / (1.0 + r)
        comb_ref[...] = (jnp.where(col == i1, w1, 0.0)
                         + jnp.where(col == i2, w2, 0.0))
        idx_ref[...] = jnp.concatenate([i1, i2], axis=1)
        xb_ref[...] = x_ref[...].astype(jnp.bfloat16)

    @pl.when(e == 0)
    def _init():
        out_ref[...] = jnp.dot(comb_ref[...], b_ref[...],
                               preferred_element_type=jnp.float32)

    ce = jnp.sum(
        comb_ref[...] * (lax.broadcasted_iota(jnp.int32, (T, E), 1) == e),
        axis=1, keepdims=True)
    acc = jnp.dot(xb_ref[...], w_ref[0].astype(jnp.bfloat16),
                  preferred_element_type=jnp.float32)
    out_ref[...] += ce * acc


@jax.jit
def kernel(x, Wg, bg, W, b):
    bg2 = bg.reshape(1, E)
    grid = (NN, E)
    out, idx = pl.pallas_call(
        _moe_dense_body,
        grid=grid,
        in_specs=[
            pl.BlockSpec((T, D), lambda n, e: (0, 0)),
            pl.BlockSpec((D, E), lambda n, e: (0, 0)),
            pl.BlockSpec((1, E), lambda n, e: (0, 0)),
            pl.BlockSpec((1, D, DN), lambda n, e: (e, 0, n)),
            pl.BlockSpec((E, DN), lambda n, e: (0, n)),
        ],
        out_specs=[
            pl.BlockSpec((T, DN), lambda n, e: (0, n)),
            pl.BlockSpec((T, TOP_K), lambda n, e: (0, 0)),
        ],
        out_shape=[
            jax.ShapeDtypeStruct((T, D), jnp.float32),
            jax.ShapeDtypeStruct((T, TOP_K), jnp.int32),
        ],
        scratch_shapes=[
            pltpu.VMEM((T, E), jnp.float32),
            pltpu.VMEM((T, D), jnp.bfloat16),
        ],
        compiler_params=pltpu.CompilerParams(
            dimension_semantics=("arbitrary", "arbitrary"),
        ),
    )(x, Wg, bg2, W, b)
    return out, idx
